# baseline (speedup 1.0000x reference)
import jax
import jax.numpy as jnp
from jax import lax
from jax.experimental import pallas as pl


def _body(ids_ref, emb_ref, w_ref, b_ref, out_ref):
    bb, ll = ids_ref.shape
    vocab = emb_ref.shape[0]
    ids3 = ids_ref[...][:, :, None]
    iota_v = lax.broadcasted_iota(jnp.int32, (bb, ll, vocab), 2)
    onehot = jnp.where(iota_v == ids3, 1.0, 0.0).astype(jnp.bfloat16)
    hidden = lax.dot_general(
        onehot, emb_ref[...],
        dimension_numbers=(((2,), (0,)), ((), ())),
        preferred_element_type=jnp.float32,
    )
    logits = lax.dot_general(
        hidden.astype(jnp.bfloat16), w_ref[...],
        dimension_numbers=(((2,), (1,)), ((), ())),
        preferred_element_type=jnp.float32,
    )
    out_ref[...] = logits + b_ref[...]


def kernel(input_ids, emb_table, lm_head_w, lm_head_b):
    B, L = input_ids.shape
    V, H = emb_table.shape
    BB = 64
    assert B % BB == 0
    bias3 = lm_head_b.reshape(1, 1, V)

    return pl.pallas_call(
        _body,
        grid=(B // BB,),
        in_specs=[
            pl.BlockSpec((BB, L), lambda i: (i, 0)),
            pl.BlockSpec((V, H), lambda i: (0, 0)),
            pl.BlockSpec((V, H), lambda i: (0, 0)),
            pl.BlockSpec((1, 1, V), lambda i: (0, 0, 0)),
        ],
        out_specs=pl.BlockSpec((BB, L, V), lambda i: (i, 0, 0)),
        out_shape=jax.ShapeDtypeStruct((B, L, V), jnp.float32),
    )(input_ids, emb_table.astype(jnp.bfloat16), lm_head_w.astype(jnp.bfloat16), bias3)
